# baseline (speedup 1.0000x reference)
import functools

import jax
import jax.numpy as jnp
from jax import lax
from jax.experimental import pallas as pl
from jax.experimental.pallas import tpu as pltpu
from jax.experimental.pallas import tpu_sc as plsc

D = 64
ROWS = 4096
COLS = 200
B = ROWS * COLS
V = 1_000_000
VP = V // 2
NC = 2
NS = 16
NW = NC * NS
SCALE = 8.0

TILES = V // 8
CT = 16
NTBASE = TILES // NW
NCHUNK = -(-(NTBASE + 2) // CT)
NCH2 = NCHUNK + (NCHUNK % 2)

LPW = B // NW
C = 128
NG = LPW // C
NGPAIR = NG // 2


def _prep(lut3d):
    mesh = plsc.VectorSubcoreMesh(core_axis_name="c", subcore_axis_name="s")

    @functools.partial(
        pl.kernel,
        mesh=mesh,
        out_type=jax.ShapeDtypeStruct((VP, 128), jnp.float32),
        scratch_types=[
            pltpu.VMEM((2, CT, 8, D), jnp.float32),
            pltpu.VMEM((2, CT * 4, 128), jnp.float32),
            pltpu.SemaphoreType.DMA,
            pltpu.SemaphoreType.DMA,
            pltpu.SemaphoreType.DMA,
            pltpu.SemaphoreType.DMA,
        ],
    )
    def k(src_hbm, tbl_hbm, inb, outb, i0, i1, o0, o1):
        wid = lax.axis_index("s") * NC + lax.axis_index("c")
        nt = NTBASE + jnp.where(wid < 4, 2, 0)
        t0 = NTBASE * wid + 2 * jnp.minimum(wid, 4)
        isem = (i0, i1)
        osem = (o0, o1)

        def tc(j):
            return t0 + jnp.minimum(j * CT, nt - CT)

        def fire_load(j, bf):
            pltpu.async_copy(src_hbm.at[pl.ds(tc(j), CT)], inb.at[bf], isem[bf])

        def wait_load(bf):
            pltpu.make_async_copy(
                src_hbm.at[pl.ds(0, CT)], inb.at[bf], isem[bf]
            ).wait()

        def scale_pack(bf):
            for i in range(CT):
                @plsc.parallel_loop(0, 4, step=1, unroll=4)
                def _(s2):
                    for q in range(D // 16):
                        sl = pl.ds(q * 16, 16)
                        sh = pl.ds(64 + q * 16, 16)
                        outb[bf, 4 * i + s2, sl] = inb[bf, i, 2 * s2, sl] * SCALE
                        outb[bf, 4 * i + s2, sh] = inb[bf, i, 2 * s2 + 1, sl] * SCALE

        def fire_store(j, bf):
            pltpu.async_copy(
                outb.at[bf], tbl_hbm.at[pl.ds(tc(j) * 4, CT * 4)], osem[bf]
            )

        def wait_store(bf):
            pltpu.make_async_copy(
                outb.at[bf], tbl_hbm.at[pl.ds(0, CT * 4)], osem[bf]
            ).wait()

        fire_load(0, 0)

        def step(t, carry):
            je = 2 * t
            wait_load(0)
            scale_pack(0)

            @pl.when(t > 0)
            def _():
                wait_store(1)

            fire_load(je + 1, 1)
            fire_store(je, 0)
            wait_load(1)
            scale_pack(1)
            wait_store(0)

            @pl.when(t < NCH2 // 2 - 1)
            def _():
                fire_load(je + 2, 0)

            fire_store(je + 1, 1)
            return carry

        lax.fori_loop(0, NCH2 // 2, step, 0)
        wait_store(1)

    return k(lut3d)


def _lookup(xf, tbl):
    mesh = plsc.VectorSubcoreMesh(core_axis_name="c", subcore_axis_name="s")

    @functools.partial(
        pl.kernel,
        mesh=mesh,
        out_type=jax.ShapeDtypeStruct((B, D), jnp.float32),
        scratch_types=[
            pltpu.VMEM((2, C), jnp.int32),
            pltpu.VMEM((2, C), jnp.int32),
            pltpu.VMEM((2, C, 128), jnp.float32),
            pltpu.VMEM((2, C, D), jnp.float32),
            pltpu.SemaphoreType.DMA,
            pltpu.SemaphoreType.DMA,
            pltpu.SemaphoreType.DMA,
            pltpu.SemaphoreType.DMA,
        ],
        compiler_params=pltpu.CompilerParams(needs_layout_passes=False),
    )
    def k(idx_hbm, tbl_hbm, out_hbm, idx_v, q_v, rows_v, pk_v, g0, g1, s0, s1):
        wid = lax.axis_index("s") * NC + lax.axis_index("c")
        base = wid * LPW
        gsem = (g0, g1)
        ssem = (s0, s1)
        lanes = lax.iota(jnp.int32, 16)

        def fire_gather(g, bf):
            r0 = base + g * C
            pltpu.sync_copy(idx_hbm.at[pl.ds(r0, C)], idx_v.at[bf])
            for kk in range(C // 16):
                sl = pl.ds(kk * 16, 16)
                q_v[bf, sl] = lax.shift_right_logical(idx_v[bf, sl], 1)
            pltpu.async_copy(tbl_hbm.at[q_v.at[bf]], rows_v.at[bf], gsem[bf])

        def wait_gather(bf):
            pltpu.make_async_copy(
                tbl_hbm.at[pl.ds(0, C)], rows_v.at[bf], gsem[bf]
            ).wait()

        def repack(bf):
            rows2d = rows_v.at[bf]
            pk2d = pk_v.at[bf]
            for kk in range(C // 16):
                sl = pl.ds(kk * 16, 16)
                rvec = lanes + (kk * 16)
                off = (idx_v[bf, sl] & 1) * D

                @plsc.parallel_loop(0, D, step=1, unroll=8)
                def _(cq):
                    cvec = lanes * 0 + cq
                    src = plsc.load_gather(rows2d, [rvec, off + cvec])
                    plsc.store_scatter(pk2d, [rvec, cvec], src)

        def start_store(g, bf):
            r0 = base + g * C
            pltpu.async_copy(pk_v.at[bf], out_hbm.at[pl.ds(r0, C)], ssem[bf])

        def wait_store(bf):
            pltpu.make_async_copy(
                pk_v.at[bf], out_hbm.at[pl.ds(0, C)], ssem[bf]
            ).wait()

        fire_gather(0, 0)

        def step(t, carry):
            ge = 2 * t
            wait_gather(0)
            repack(0)

            @pl.when(t > 0)
            def _():
                wait_store(1)

            fire_gather(ge + 1, 1)
            start_store(ge, 0)
            wait_gather(1)
            repack(1)
            wait_store(0)

            @pl.when(t < NGPAIR - 1)
            def _():
                fire_gather(ge + 2, 0)

            start_store(ge + 1, 1)
            return carry

        lax.fori_loop(0, NGPAIR, step, 0)
        wait_store(1)

    return k(xf, tbl)


def kernel(x, lut):
    xf = x.reshape(B)
    lut3d = lut.reshape(TILES, 8, D)
    tbl = _prep(lut3d)
    out = _lookup(xf, tbl)
    return out.reshape(ROWS, COLS, D)
